# baseline (speedup 1.0000x reference)
import jax
import jax.numpy as jnp
from jax import lax
from jax.experimental import pallas as pl
from jax.experimental.pallas import tpu as pltpu
from jax.experimental.pallas import tpu_sc as plsc

_NUM_CLASSES = 1000000
_HIDDEN = 64
_BATCH = 16384

_NC = 2
_NS = 16
_LANES = 16
_NW = _NC * _NS
_BPW = _BATCH // _NW
_CHUNK = 128
_NCHUNK = _BPW // _CHUNK
_JH = _HIDDEN // _LANES


def _emb_kernel(labels_hbm, drops_hbm, tpair_hbm, dr_hbm, out_hbm,
                lab_v, drops_v, pidx_v, dr_v, prow_v, outrows_v, gsem, dsem):
    wid = lax.axis_index("s") * _NC + lax.axis_index("c")
    base = wid * _BPW

    pltpu.sync_copy(labels_hbm.at[wid], lab_v)
    pltpu.sync_copy(drops_hbm.at[wid], drops_v)
    pltpu.sync_copy(dr_hbm, dr_v)

    for c in range(_BPW // _LANES):
        l = lab_v[c // 8, pl.ds((c % 8) * _LANES, _LANES)]
        pidx_v[c // 8, pl.ds((c % 8) * _LANES, _LANES)] = l >> 1

    drj = [dr_v[0, pl.ds(j * _LANES, _LANES)] for j in range(_JH)]

    def fire(j, slot):
        return pltpu.async_copy(tpair_hbm.at[pidx_v.at[j]],
                                prow_v.at[slot], gsem)

    cp = fire(0, 0)
    for j in range(_NCHUNK):
        nxt = fire(j + 1, (j + 1) % 2) if j + 1 < _NCHUNK else None
        cp.wait()
        slot = j % 2

        def asm(g, carry):
            off = g * _LANES
            lvec = lab_v[j, pl.ds(off, _LANES)]
            dvec = drops_v[j, pl.ds(off, _LANES)]
            for k in range(_LANES):
                row = prow_v.at[slot].at[off + k]
                orow = outrows_v.at[slot].at[off + k]
                h = (lvec[k] & 1) * _HIDDEN
                d = dvec[k]
                for jj in range(_JH):
                    val = row[pl.ds(h + jj * _LANES, _LANES)]
                    orow[pl.ds(jj * _LANES, _LANES)] = jnp.where(
                        d == 1, drj[jj], val)
            return carry

        lax.fori_loop(0, _CHUNK // _LANES, asm, 0)
        pltpu.sync_copy(outrows_v.at[slot],
                        out_hbm.at[pl.ds(base + j * _CHUNK, _CHUNK)])
        cp = nxt


@jax.jit
def _embed(labels, drops, tpair, dr):
    mesh = plsc.VectorSubcoreMesh(core_axis_name="c", subcore_axis_name="s")
    return pl.kernel(
        _emb_kernel,
        mesh=mesh,
        out_type=jax.ShapeDtypeStruct((_BATCH, _HIDDEN), jnp.float32),
        scratch_types=[
            pltpu.VMEM((_NCHUNK, _CHUNK), jnp.int32),
            pltpu.VMEM((_NCHUNK, _CHUNK), jnp.int32),
            pltpu.VMEM((_NCHUNK, _CHUNK), jnp.int32),
            pltpu.VMEM((1, _HIDDEN), jnp.float32),
            pltpu.VMEM((2, _CHUNK, 2 * _HIDDEN), jnp.float32),
            pltpu.VMEM((2, _CHUNK, _HIDDEN), jnp.float32),
            pltpu.SemaphoreType.DMA,
            pltpu.SemaphoreType.DMA,
        ],
    )(labels, drops, tpair, dr)


def kernel(class_labels, train, force_drop_ids, table):
    del train
    labels3 = class_labels.astype(jnp.int32).reshape(_NW, _NCHUNK, _CHUNK)
    drops3 = force_drop_ids.astype(jnp.int32).reshape(_NW, _NCHUNK, _CHUNK)
    dr = table[_NUM_CLASSES].reshape(1, _HIDDEN)
    tpair = table[:_NUM_CLASSES].reshape(_NUM_CLASSES // 2, 2 * _HIDDEN)
    return _embed(labels3, drops3, tpair, dr)
